# baseline (speedup 1.0000x reference)
import functools

import jax
import jax.numpy as jnp
from jax import lax
from jax.experimental import pallas as pl
from jax.experimental.pallas import tpu as pltpu
from jax.experimental.pallas import tpu_sc as plsc

N_ATOMS = 1_000_000
B_TOTAL = 16384

NC = 2
NS = 16
NW = NC * NS
LANES = 16

CH = 31_232
CH_LAST = N_ATOMS - (NW - 1) * CH
PADSLOT = CH_LAST
VBUF = CH_LAST + LANES

GROUPS = B_TOTAL // LANES
UNROLL = 16

_mesh = plsc.VectorSubcoreMesh(
    core_axis_name="c", subcore_axis_name="s", num_cores=NC, num_subcores=NS
)


@functools.partial(
    pl.kernel,
    out_type=jax.ShapeDtypeStruct((1, N_ATOMS), jnp.float32),
    mesh=_mesh,
    compiler_params=pltpu.CompilerParams(needs_layout_passes=False),
    scratch_types=[
        pltpu.VMEM((VBUF,), jnp.float32),
        pltpu.VMEM((B_TOTAL,), jnp.int32),
        pltpu.VMEM((B_TOTAL,), jnp.float32),
    ],
)
def _facts_scatter(v0_hbm, idx_hbm, val_hbm, out_hbm, vbuf, idxb, valb):
    c = lax.axis_index("c")
    s = lax.axis_index("s")
    w = c * NS + s
    woff = pl.multiple_of(w * CH, 128)
    wsize = jnp.where(w == NW - 1, CH_LAST, CH)

    @pl.when(w < NW - 1)
    def _init_main():
        pltpu.sync_copy(v0_hbm.at[0, pl.ds(woff, CH)], vbuf.at[pl.ds(0, CH)])

    @pl.when(w == NW - 1)
    def _init_last():
        pltpu.sync_copy(v0_hbm.at[0, pl.ds((NW - 1) * CH, CH_LAST)],
                        vbuf.at[pl.ds(0, CH_LAST)])

    pltpu.sync_copy(idx_hbm, idxb)
    pltpu.sync_copy(val_hbm, valb)

    @pl.when(w == 0)
    def _bias():
        lane = lax.iota(jnp.int32, LANES)
        head = vbuf[pl.ds(0, LANES)]
        vbuf[pl.ds(0, LANES)] = head + jnp.where(lane == 0, 1.0, 0.0).astype(
            jnp.float32)

    def _body(it, carry):
        st0 = it * (UNROLL * LANES)
        for u in range(UNROLL):
            st = st0 + u * LANES
            g = idxb[pl.ds(st, LANES)]
            local = g - woff
            inb = (local >= 0) & (local < wsize)
            lsafe = jnp.where(inb, local, PADSLOT)
            v = valb[pl.ds(st, LANES)]
            plsc.addupdate_scatter(vbuf, [lsafe], v)
        return carry

    lax.fori_loop(0, GROUPS // UNROLL, _body, 0, unroll=False)

    @pl.when(w < NW - 1)
    def _wb_main():
        pltpu.sync_copy(vbuf.at[pl.ds(0, CH)], out_hbm.at[0, pl.ds(woff, CH)])

    @pl.when(w == NW - 1)
    def _wb_last():
        pltpu.sync_copy(vbuf.at[pl.ds(0, CH_LAST)],
                        out_hbm.at[0, pl.ds((NW - 1) * CH, CH_LAST)])


@jax.jit
def kernel(V0, val, bk_idx):
    idx = bk_idx.astype(jnp.int32)
    vals = val.astype(jnp.float32)
    return _facts_scatter(V0, idx, vals)


# speedup vs baseline: 1.3562x; 1.3562x over previous
import functools

import jax
import jax.numpy as jnp
from jax import lax
from jax.experimental import pallas as pl
from jax.experimental.pallas import tpu as pltpu
from jax.experimental.pallas import tpu_sc as plsc

N_ATOMS = 1_000_000
B_TOTAL = 16384

NC = 2
NS = 16
LANES = 16

H0 = 500_096
H1 = N_ATOMS - H0
DUMP = H0
SP_WORDS = H0 + 128

CHUNK = B_TOTAL // NS
ROWS = 8
COLS = 128
assert ROWS * COLS == CHUNK

CH = 31_232
CH0_LAST = H0 - 15 * CH
CH1_LAST = H1 - 15 * CH
BUF = max(CH, CH0_LAST, CH1_LAST)

_mesh = plsc.VectorSubcoreMesh(
    core_axis_name="c", subcore_axis_name="s", num_cores=NC, num_subcores=NS
)


@functools.partial(
    pl.kernel,
    out_type=jax.ShapeDtypeStruct((1, N_ATOMS), jnp.float32),
    mesh=_mesh,
    scratch_types=[
        pltpu.VMEM_SHARED((SP_WORDS,), jnp.float32),
        pltpu.VMEM((ROWS, COLS), jnp.int32),
        pltpu.VMEM((ROWS, COLS), jnp.int32),
        pltpu.VMEM((ROWS, COLS), jnp.float32),
        pltpu.VMEM((LANES,), jnp.int32),
        pltpu.VMEM((LANES,), jnp.float32),
        pltpu.VMEM((BUF,), jnp.float32),
        pltpu.VMEM((COLS,), jnp.float32),
        pltpu.SemaphoreType.DMA,
    ],
)
def _facts_scatter(v0_hbm, idx_hbm, val_hbm, out_hbm,
                   vsh, idx_raw, idx_loc, vals, bidx, bval, bufa, zbuf, sem_sc):
    c = lax.axis_index("c")
    s = lax.axis_index("s")
    base = c * H0
    hsize = H0 - c * (H0 - H1)
    off = pl.multiple_of(s * CH, 128)

    def _fire_init(n, hoff):
        pltpu.sync_copy(v0_hbm.at[0, pl.ds(hoff, n)], bufa.at[pl.ds(0, n)])
        pltpu.sync_copy(bufa.at[pl.ds(0, n)], vsh.at[pl.ds(off, n)])

    @pl.when(s < NS - 1)
    def _init_main():
        _fire_init(CH, base + off)

    @pl.when((s == NS - 1) & (c == 0))
    def _init_last0():
        _fire_init(CH0_LAST, 15 * CH)

    @pl.when((s == NS - 1) & (c == 1))
    def _init_last1():
        _fire_init(CH1_LAST, H0 + 15 * CH)

    pltpu.sync_copy(idx_hbm.at[s], idx_raw)
    pltpu.sync_copy(val_hbm.at[s], vals)

    for r in range(ROWS):
        for k in range(COLS // LANES):
            g = idx_raw[r, pl.ds(k * LANES, LANES)]
            local = g - base
            in_range = (local >= 0) & (local < hsize)
            idx_loc[r, pl.ds(k * LANES, LANES)] = jnp.where(in_range, local, DUMP)

    lane = lax.iota(jnp.int32, LANES)
    bidx[...] = jnp.where(lane == 0, 0, DUMP)
    bval[...] = jnp.where(lane == 0, 1.0, 0.0).astype(jnp.float32)
    for k in range(COLS // LANES):
        zbuf[pl.ds(k * LANES, LANES)] = jnp.zeros((LANES,), jnp.float32)

    plsc.subcore_barrier()

    descs = [
        pltpu.async_copy(vals.at[r], vsh.at[idx_loc.at[r]], sem_sc, add=True)
        for r in range(ROWS)
    ]
    for d in descs:
        d.wait()

    @pl.when((c == 0) & (s == 0))
    def _bias():
        pltpu.sync_copy(bval, vsh.at[bidx], add=True)

    pltpu.sync_copy(zbuf, vsh.at[idx_loc.at[ROWS - 1]], add=True)
    pltpu.sync_copy(zbuf.at[pl.ds(0, LANES)], vsh.at[bidx], add=True)

    plsc.subcore_barrier()

    def _writeback(n, hoff):
        pltpu.sync_copy(vsh.at[pl.ds(off, n)], bufa.at[pl.ds(0, n)])
        pltpu.sync_copy(bufa.at[pl.ds(0, n)], out_hbm.at[0, pl.ds(hoff, n)])

    @pl.when(s < NS - 1)
    def _wb_main():
        _writeback(CH, base + off)

    @pl.when((s == NS - 1) & (c == 0))
    def _wb_last0():
        _writeback(CH0_LAST, 15 * CH)

    @pl.when((s == NS - 1) & (c == 1))
    def _wb_last1():
        _writeback(CH1_LAST, H0 + 15 * CH)


@jax.jit
def kernel(V0, val, bk_idx):
    idx = bk_idx.astype(jnp.int32).reshape(NS, ROWS, COLS)
    vals = val.astype(jnp.float32).reshape(NS, ROWS, COLS)
    return _facts_scatter(V0, idx, vals)


# speedup vs baseline: 1.3695x; 1.0097x over previous
import functools

import jax
import jax.numpy as jnp
from jax import lax
from jax.experimental import pallas as pl
from jax.experimental.pallas import tpu as pltpu
from jax.experimental.pallas import tpu_sc as plsc

N_ATOMS = 1_000_000
B_TOTAL = 16384

NC = 2
NS = 16
LANES = 16

H0 = 500_096
H1 = N_ATOMS - H0
DUMP = H0
SP_WORDS = H0 + 128

CHUNK = B_TOTAL // NS
ROWS = 8
COLS = 128
assert ROWS * COLS == CHUNK

CH = 31_232
CH0_LAST = H0 - 15 * CH
CH1_LAST = H1 - 15 * CH
BUF = max(CH, CH0_LAST, CH1_LAST)

_mesh = plsc.VectorSubcoreMesh(
    core_axis_name="c", subcore_axis_name="s", num_cores=NC, num_subcores=NS
)


@functools.partial(
    pl.kernel,
    out_type=jax.ShapeDtypeStruct((1, N_ATOMS), jnp.float32),
    mesh=_mesh,
    scratch_types=[
        pltpu.VMEM_SHARED((SP_WORDS,), jnp.float32),
        pltpu.VMEM((ROWS, COLS), jnp.int32),
        pltpu.VMEM((ROWS, COLS), jnp.int32),
        pltpu.VMEM((ROWS, COLS), jnp.float32),
        pltpu.VMEM((BUF,), jnp.float32),
        pltpu.VMEM((COLS,), jnp.float32),
    ],
)
def _facts_scatter(v0_hbm, idx_hbm, val_hbm, out_hbm,
                   vsh, idx_raw, idx_loc, vals, bufa, zbuf):
    c = lax.axis_index("c")
    s = lax.axis_index("s")
    base = c * H0
    hsize = H0 - c * (H0 - H1)
    off = pl.multiple_of(s * CH, 128)

    def _fire_init(n, hoff):
        pltpu.sync_copy(v0_hbm.at[0, pl.ds(hoff, n)], bufa.at[pl.ds(0, n)])
        pltpu.sync_copy(bufa.at[pl.ds(0, n)], vsh.at[pl.ds(off, n)])

    @pl.when(s < NS - 1)
    def _init_main():
        _fire_init(CH, base + off)

    @pl.when((s == NS - 1) & (c == 0))
    def _init_last0():
        _fire_init(CH0_LAST, 15 * CH)

    @pl.when((s == NS - 1) & (c == 1))
    def _init_last1():
        _fire_init(CH1_LAST, H0 + 15 * CH)

    pltpu.sync_copy(idx_hbm.at[s], idx_raw)
    pltpu.sync_copy(val_hbm.at[s], vals)

    for r in range(ROWS):
        for k in range(COLS // LANES):
            g = idx_raw[r, pl.ds(k * LANES, LANES)]
            local = g - base
            in_range = (local >= 0) & (local < hsize)
            idx_loc[r, pl.ds(k * LANES, LANES)] = jnp.where(in_range, local, DUMP)

    for k in range(COLS // LANES):
        zbuf[pl.ds(k * LANES, LANES)] = jnp.zeros((LANES,), jnp.float32)

    plsc.subcore_barrier()

    for r in range(ROWS):
        pltpu.sync_copy(vals.at[r], vsh.at[idx_loc.at[r]], add=True)

    pltpu.sync_copy(zbuf, vsh.at[idx_loc.at[ROWS - 1]], add=True)

    plsc.subcore_barrier()

    def _writeback(n, hoff):
        pltpu.sync_copy(vsh.at[pl.ds(off, n)], bufa.at[pl.ds(0, n)])
        @pl.when((c == 0) & (s == 0))
        def _bias():
            lane = lax.iota(jnp.int32, LANES)
            head = bufa[pl.ds(0, LANES)]
            bufa[pl.ds(0, LANES)] = head + jnp.where(
                lane == 0, 1.0, 0.0).astype(jnp.float32)
        pltpu.sync_copy(bufa.at[pl.ds(0, n)], out_hbm.at[0, pl.ds(hoff, n)])

    @pl.when(s < NS - 1)
    def _wb_main():
        _writeback(CH, base + off)

    @pl.when((s == NS - 1) & (c == 0))
    def _wb_last0():
        _writeback(CH0_LAST, 15 * CH)

    @pl.when((s == NS - 1) & (c == 1))
    def _wb_last1():
        _writeback(CH1_LAST, H0 + 15 * CH)


@jax.jit
def kernel(V0, val, bk_idx):
    idx = bk_idx.astype(jnp.int32).reshape(NS, ROWS, COLS)
    vals = val.astype(jnp.float32).reshape(NS, ROWS, COLS)
    return _facts_scatter(V0, idx, vals)


# speedup vs baseline: 1.6722x; 1.2211x over previous
import functools

import jax
import jax.numpy as jnp
from jax import lax
from jax.experimental import pallas as pl
from jax.experimental.pallas import tpu as pltpu
from jax.experimental.pallas import tpu_sc as plsc

N_ATOMS = 1_000_000
B_TOTAL = 16384

NC = 2
NS = 16
LANES = 16

H0 = 500_096
H1 = N_ATOMS - H0
DUMP = H0
SP_WORDS = H0 + 128

CHUNK = B_TOTAL // NS
ROWS = 8
COLS = 128
assert ROWS * COLS == CHUNK

CH = 31_232
CH0_LAST = H0 - 15 * CH
CH1_LAST = H1 - 15 * CH
BUF = max(CH, CH0_LAST, CH1_LAST)

_mesh = plsc.VectorSubcoreMesh(
    core_axis_name="c", subcore_axis_name="s", num_cores=NC, num_subcores=NS
)


@functools.partial(
    pl.kernel,
    out_type=jax.ShapeDtypeStruct((1, N_ATOMS), jnp.float32),
    mesh=_mesh,
    compiler_params=pltpu.CompilerParams(needs_layout_passes=False),
    scratch_types=[
        pltpu.VMEM_SHARED((SP_WORDS,), jnp.float32),
        pltpu.VMEM((ROWS, COLS), jnp.int32),
        pltpu.VMEM((CHUNK + LANES,), jnp.int32),
        pltpu.VMEM((ROWS, COLS), jnp.float32),
        pltpu.VMEM((CHUNK + LANES,), jnp.float32),
        pltpu.VMEM((BUF,), jnp.float32),
        pltpu.VMEM((COLS,), jnp.float32),
    ],
)
def _facts_scatter(v0_hbm, idx_hbm, val_hbm, out_hbm,
                   vsh, idx_raw, idxc, vals, valc, bufa, zbuf):
    c = lax.axis_index("c")
    s = lax.axis_index("s")
    base = c * H0
    hsize = H0 - c * (H0 - H1)
    off = pl.multiple_of(s * CH, 128)

    def _fire_init(n, hoff):
        pltpu.sync_copy(v0_hbm.at[0, pl.ds(hoff, n)], bufa.at[pl.ds(0, n)])
        pltpu.sync_copy(bufa.at[pl.ds(0, n)], vsh.at[pl.ds(off, n)])

    @pl.when(s < NS - 1)
    def _init_main():
        _fire_init(CH, base + off)

    @pl.when((s == NS - 1) & (c == 0))
    def _init_last0():
        _fire_init(CH0_LAST, 15 * CH)

    @pl.when((s == NS - 1) & (c == 1))
    def _init_last1():
        _fire_init(CH1_LAST, H0 + 15 * CH)

    pltpu.sync_copy(idx_hbm.at[s], idx_raw)
    pltpu.sync_copy(val_hbm.at[s], vals)

    dump_vec = jnp.full((LANES,), DUMP, jnp.int32)
    zero_vec = jnp.zeros((LANES,), jnp.float32)
    for k in range(CHUNK // LANES):
        idxc[pl.ds(k * LANES, LANES)] = dump_vec
        valc[pl.ds(k * LANES, LANES)] = zero_vec
    for k in range(COLS // LANES):
        zbuf[pl.ds(k * LANES, LANES)] = jnp.zeros((LANES,), jnp.float32)

    cur = jnp.int32(0)
    for r in range(ROWS):
        for k in range(COLS // LANES):
            g = idx_raw[r, pl.ds(k * LANES, LANES)]
            local = g - base
            in_range = (local >= 0) & (local < hsize)
            plsc.store_compressed(idxc.at[pl.ds(cur, LANES)], local,
                                  mask=in_range)
            plsc.store_compressed(valc.at[pl.ds(cur, LANES)],
                                  vals[r, pl.ds(k * LANES, LANES)],
                                  mask=in_range)
            cur = cur + plsc.all_reduce_population_count(in_range)[0]

    plsc.subcore_barrier()

    for r in range(ROWS):
        @pl.when(cur > r * COLS)
        def _scat(r=r):
            pltpu.sync_copy(valc.at[pl.ds(r * COLS, COLS)],
                            vsh.at[idxc.at[pl.ds(r * COLS, COLS)]], add=True)

    pltpu.sync_copy(zbuf, vsh.at[idxc.at[pl.ds(0, COLS)]], add=True)

    plsc.subcore_barrier()

    def _writeback(n, hoff):
        pltpu.sync_copy(vsh.at[pl.ds(off, n)], bufa.at[pl.ds(0, n)])
        @pl.when((c == 0) & (s == 0))
        def _bias():
            lane = lax.iota(jnp.int32, LANES)
            head = bufa[pl.ds(0, LANES)]
            bufa[pl.ds(0, LANES)] = head + jnp.where(
                lane == 0, 1.0, 0.0).astype(jnp.float32)
        pltpu.sync_copy(bufa.at[pl.ds(0, n)], out_hbm.at[0, pl.ds(hoff, n)])

    @pl.when(s < NS - 1)
    def _wb_main():
        _writeback(CH, base + off)

    @pl.when((s == NS - 1) & (c == 0))
    def _wb_last0():
        _writeback(CH0_LAST, 15 * CH)

    @pl.when((s == NS - 1) & (c == 1))
    def _wb_last1():
        _writeback(CH1_LAST, H0 + 15 * CH)


@jax.jit
def kernel(V0, val, bk_idx):
    idx = bk_idx.astype(jnp.int32).reshape(NS, ROWS, COLS)
    vals = val.astype(jnp.float32).reshape(NS, ROWS, COLS)
    return _facts_scatter(V0, idx, vals)
